# baseline (speedup 1.0000x reference)
import jax
import jax.numpy as jnp
from jax import lax
from jax.experimental import pallas as pl
from jax.experimental.pallas import tpu as pltpu
from jax.experimental.pallas import tpu_sc as plsc

NC, NS, L = 2, 16, 16
NW = NC * NS
B, S, D = 4096, 50, 64
BPW = B // NW
CH = 2
ROWS = CH * S
NCHUNK = BPW // CH
ND = D // L
NBUF = 16
INV_S = 1.0 / S


def _pool_body(idx_hbm, table_hbm, out_hbm, idx_v, buf, out_v, *sems):
    wid = lax.axis_index("c") * NS + lax.axis_index("s")
    pltpu.sync_copy(idx_hbm.at[wid], idx_v)

    def start(c, b):
        pltpu.async_copy(table_hbm.at[idx_v.at[c]], buf.at[b], sems[b])

    def wait(b):
        pltpu.make_async_copy(table_hbm.at[idx_v.at[0]], buf.at[b], sems[b]).wait()

    def accum(c, b):
        zero = jnp.zeros((L,), jnp.float32)

        def step(it, acc):
            s0 = it * 10
            acc = list(acc)
            for k in range(10):
                for e in range(CH):
                    for d in range(ND):
                        acc[e * ND + d] = acc[e * ND + d] + buf[
                            b, e * S + s0 + k, pl.ds(d * L, L)
                        ]
            return tuple(acc)

        acc = lax.fori_loop(0, S // 10, step, (zero,) * (CH * ND))
        for e in range(CH):
            for d in range(ND):
                out_v[c * CH + e, pl.ds(d * L, L)] = (
                    acc[e * ND + d] * jnp.float32(INV_S)
                )

    for p in range(NBUF - 1):
        start(p, p)

    def loop_body(t, carry):
        for b in range(NBUF):
            c = t * NBUF + b
            wait(b)
            start(c + NBUF - 1, (b + NBUF - 1) % NBUF)
            accum(c, b)
        return carry

    lax.fori_loop(0, NCHUNK // NBUF - 1, loop_body, 0)
    for c in range(NCHUNK - NCHUNK % NBUF - NBUF, NCHUNK):
        b = c % NBUF
        wait(b)
        if c + NBUF - 1 < NCHUNK:
            start(c + NBUF - 1, (b + NBUF - 1) % NBUF)
        accum(c, b)

    pltpu.sync_copy(out_v, out_hbm.at[pl.ds(wid * BPW, BPW)])


def kernel(inputs, cvm, table_pri, table_sec):
    del cvm, table_sec
    idx = inputs.astype(jnp.int32).reshape(NW, NCHUNK, ROWS)
    run = pl.kernel(
        _pool_body,
        out_type=jax.ShapeDtypeStruct((B, D), jnp.float32),
        mesh=plsc.VectorSubcoreMesh(core_axis_name="c", subcore_axis_name="s"),
        scratch_types=[
            pltpu.VMEM((NCHUNK, ROWS), jnp.int32),
            pltpu.VMEM((NBUF, ROWS, D), jnp.float32),
            pltpu.VMEM((BPW, D), jnp.float32),
        ] + [pltpu.SemaphoreType.DMA] * NBUF,
        compiler_params=pltpu.CompilerParams(use_tc_tiling_on_sc=False),
    )
    return run(idx, table_pri)


# speedup vs baseline: 1.0178x; 1.0178x over previous
import jax
import jax.numpy as jnp
from jax import lax
from jax.experimental import pallas as pl
from jax.experimental.pallas import tpu as pltpu
from jax.experimental.pallas import tpu_sc as plsc

NC, NS, L = 2, 16, 16
NW = NC * NS
B, S, D = 4096, 50, 64
BPW = B // NW
CH = 2
ROWS = CH * S
NCHUNK = BPW // CH
ND = D // L
NBUF = 8
INV_S = 1.0 / S


def _pool_body(idx_hbm, table_hbm, out_hbm, idx_v, buf, out_v, *sems):
    wid = lax.axis_index("c") * NS + lax.axis_index("s")
    pltpu.sync_copy(idx_hbm.at[wid], idx_v)

    def start(c, b):
        pltpu.async_copy(table_hbm.at[idx_v.at[c]], buf.at[b], sems[b])

    def wait(b):
        pltpu.make_async_copy(table_hbm.at[idx_v.at[0]], buf.at[b], sems[b]).wait()

    def accum(c, b):
        zero = jnp.zeros((L,), jnp.float32)

        def step(it, acc):
            s0 = it * 10
            acc = list(acc)
            for k in range(10):
                for e in range(CH):
                    for d in range(ND):
                        acc[e * ND + d] = acc[e * ND + d] + buf[
                            b, e * S + s0 + k, pl.ds(d * L, L)
                        ]
            return tuple(acc)

        acc = lax.fori_loop(0, S // 10, step, (zero,) * (CH * ND))
        for e in range(CH):
            for d in range(ND):
                out_v[c * CH + e, pl.ds(d * L, L)] = (
                    acc[e * ND + d] * jnp.float32(INV_S)
                )

    for p in range(NBUF - 1):
        start(p, p)

    def loop_body(t, carry):
        for b in range(NBUF):
            c = t * NBUF + b
            wait(b)
            start(c + NBUF - 1, (b + NBUF - 1) % NBUF)
            accum(c, b)
        return carry

    lax.fori_loop(0, NCHUNK // NBUF - 1, loop_body, 0)
    for c in range(NCHUNK - NCHUNK % NBUF - NBUF, NCHUNK):
        b = c % NBUF
        wait(b)
        if c + NBUF - 1 < NCHUNK:
            start(c + NBUF - 1, (b + NBUF - 1) % NBUF)
        accum(c, b)

    pltpu.sync_copy(out_v, out_hbm.at[pl.ds(wid * BPW, BPW)])


def kernel(inputs, cvm, table_pri, table_sec):
    del cvm, table_sec
    idx = inputs.astype(jnp.int32).reshape(NW, NCHUNK, ROWS)
    run = pl.kernel(
        _pool_body,
        out_type=jax.ShapeDtypeStruct((B, D), jnp.float32),
        mesh=plsc.VectorSubcoreMesh(core_axis_name="c", subcore_axis_name="s"),
        scratch_types=[
            pltpu.VMEM((NCHUNK, ROWS), jnp.int32),
            pltpu.VMEM((NBUF, ROWS, D), jnp.float32),
            pltpu.VMEM((BPW, D), jnp.float32),
        ] + [pltpu.SemaphoreType.DMA] * NBUF,
        compiler_params=pltpu.CompilerParams(use_tc_tiling_on_sc=False),
    )
    return run(idx, table_pri)
